# baseline (speedup 1.0000x reference)
import functools

import jax
import jax.numpy as jnp
from jax import lax
from jax.experimental import pallas as pl
from jax.experimental.pallas import tpu as pltpu
from jax.experimental.pallas import tpu_sc as plsc

N = 50000
F = 512
B = 256
L = 16
NC = 2
NS = 16
CH = 2000
NCH = N // CH
U = 16


def _pool_body(x_hbm, ids_hbm, c_hbm, out_hbm,
               xbuf, idbuf, cbuf, acc_e, acc_w, acc_c, obuf, semx, semi):
    cid = lax.axis_index("c")
    sid = lax.axis_index("s")
    w = sid * NC + cid
    col0 = w * L

    pltpu.sync_copy(c_hbm, cbuf)
    p_vec = cbuf[0, :]
    beta_vec = cbuf[1, :]
    scale_vec = cbuf[2, :]
    zeros = jnp.zeros((L,), jnp.float32)
    ones = jnp.ones((L,), jnp.float32)
    sixteen = jnp.full((L,), float(U), jnp.float32)

    def zero_body(b, carry):
        off = b * L
        acc_e[pl.ds(off, L)] = zeros
        acc_w[pl.ds(off, L)] = zeros
        acc_c[pl.ds(off, L)] = zeros
        return carry

    lax.fori_loop(0, B, zero_body, 0)

    def copies(ci):
        s = lax.rem(ci, 2)
        r0 = ci * CH
        cx = pltpu.make_async_copy(
            x_hbm.at[pl.ds(r0, CH), pl.ds(col0, L)], xbuf.at[s], semx.at[s])
        cb = pltpu.make_async_copy(
            ids_hbm.at[pl.ds(r0, CH)], idbuf.at[pl.ds(s * CH, CH)], semi.at[s])
        return cx, cb

    cx0, cb0 = copies(0)
    cx0.start()
    cb0.start()

    def chunk_body(ci, carry):
        s = lax.rem(ci, 2)
        cx, cb = copies(ci)
        cx.wait()
        cb.wait()

        @pl.when(ci + 1 < NCH)
        def _prefetch():
            nx, nb = copies(ci + 1)
            nx.start()
            nb.start()

        def group(base):
            idv = idbuf[pl.ds(s * CH + base, U)]
            xs = [xbuf[s, base + u, :] for u in range(U)]
            es = [jnp.exp(xv * p_vec) for xv in xs]
            ws = [xs[u] * es[u] for u in range(U)]
            esum, wsum = es, ws
            while len(esum) > 1:
                esum = [a + b for a, b in zip(esum[::2], esum[1::2])]
                wsum = [a + b for a, b in zip(wsum[::2], wsum[1::2])]

            @pl.when(idv[0] == idv[U - 1])
            def _fast():
                off = idv[0] * L
                plsc.addupdate(acc_e.at[pl.ds(off, L)], esum[0])
                plsc.addupdate(acc_w.at[pl.ds(off, L)], wsum[0])
                plsc.addupdate(acc_c.at[pl.ds(off, L)], sixteen)

            @pl.when(idv[0] != idv[U - 1])
            def _slow():
                for u in range(U):
                    off = idv[u] * L
                    plsc.addupdate(acc_e.at[pl.ds(off, L)], es[u])
                    plsc.addupdate(acc_w.at[pl.ds(off, L)], ws[u])
                    plsc.addupdate(acc_c.at[pl.ds(off, L)], ones)

        def row_body(jj, inner):
            base = jj * (3 * U)
            group(base)
            group(base + U)
            group(base + 2 * U)
            return inner

        lax.fori_loop(0, CH // (3 * U), row_body, 0)
        group(CH - 2 * U)
        group(CH - U)
        return carry

    lax.fori_loop(0, NCH, chunk_body, 0)

    def fin_body(b, carry):
        off = b * L
        e = acc_e[pl.ds(off, L)]
        wv = acc_w[pl.ds(off, L)]
        n = acc_c[pl.ds(off, L)]
        pooled = wv / e
        out_v = pooled * n / (1.0 + beta_vec * (n - 1.0)) * scale_vec
        obuf[b, :] = out_v
        return carry

    lax.fori_loop(0, B, fin_body, 0)
    pltpu.sync_copy(obuf, out_hbm.at[:, pl.ds(col0, L)])


@jax.jit
def _pool(x, ids, consts):
    mesh = plsc.VectorSubcoreMesh(core_axis_name="c", subcore_axis_name="s")
    f = functools.partial(
        pl.kernel,
        out_type=jax.ShapeDtypeStruct((B, F), jnp.float32),
        mesh=mesh,
        compiler_params=pltpu.CompilerParams(
            use_tc_tiling_on_sc=False, needs_layout_passes=False),
        scratch_types=[
            pltpu.VMEM((2, CH, L), jnp.float32),
            pltpu.VMEM((2 * CH,), jnp.int32),
            pltpu.VMEM((3, L), jnp.float32),
            pltpu.VMEM((B * L,), jnp.float32),
            pltpu.VMEM((B * L,), jnp.float32),
            pltpu.VMEM((B * L,), jnp.float32),
            pltpu.VMEM((B, L), jnp.float32),
            pltpu.SemaphoreType.DMA((2,)),
            pltpu.SemaphoreType.DMA((2,)),
        ],
    )(_pool_body)
    return f(x, ids, consts)


def kernel(x, batch, p, beta, bsize):
    scale = jnp.asarray(bsize, jnp.float32) / jnp.float32(B)
    consts = jnp.stack([
        jnp.broadcast_to(p.astype(jnp.float32), (L,)),
        jnp.broadcast_to(beta.astype(jnp.float32), (L,)),
        jnp.broadcast_to(scale, (L,)),
    ])
    return _pool(x, batch.astype(jnp.int32), consts)


# speedup vs baseline: 1.0051x; 1.0051x over previous
import functools

import jax
import jax.numpy as jnp
from jax import lax
from jax.experimental import pallas as pl
from jax.experimental.pallas import tpu as pltpu
from jax.experimental.pallas import tpu_sc as plsc

N = 50000
F = 512
B = 256
L = 16
NC = 2
NS = 16
CH = 2000
NCH = N // CH
U = 16


def _pool_body(x_hbm, ids_hbm, c_hbm, out_hbm,
               xbuf, idbuf, cbuf, acc_e, acc_w, acc_c, obuf, semx, semi):
    cid = lax.axis_index("c")
    sid = lax.axis_index("s")
    w = sid * NC + cid
    col0 = w * L

    pltpu.sync_copy(c_hbm, cbuf)
    p_vec = cbuf[0, :]
    beta_vec = cbuf[1, :]
    scale_vec = cbuf[2, :]
    zeros = jnp.zeros((L,), jnp.float32)
    ones = jnp.ones((L,), jnp.float32)
    sixteen = jnp.full((L,), float(U), jnp.float32)

    def zero_body(b, carry):
        off = b * L
        acc_e[pl.ds(off, L)] = zeros
        acc_w[pl.ds(off, L)] = zeros
        acc_c[pl.ds(off, L)] = zeros
        return carry

    lax.fori_loop(0, B, zero_body, 0)

    def copies(ci):
        s = lax.rem(ci, 2)
        r0 = ci * CH
        cx = pltpu.make_async_copy(
            x_hbm.at[pl.ds(r0, CH), pl.ds(col0, L)], xbuf.at[s], semx.at[s])
        cb = pltpu.make_async_copy(
            ids_hbm.at[pl.ds(r0, CH)], idbuf.at[pl.ds(s * CH, CH)], semi.at[s])
        return cx, cb

    cx0, cb0 = copies(0)
    cx0.start()
    cb0.start()

    def chunk_body(ci, carry):
        s = lax.rem(ci, 2)
        cx, cb = copies(ci)
        cx.wait()
        cb.wait()

        @pl.when(ci + 1 < NCH)
        def _prefetch():
            nx, nb = copies(ci + 1)
            nx.start()
            nb.start()

        def group(base):
            idv = idbuf[pl.ds(s * CH + base, U)]
            xs = [xbuf[s, base + u, :] for u in range(U)]
            es = [jnp.exp(xv * p_vec) for xv in xs]
            ws = [xs[u] * es[u] for u in range(U)]
            esum, wsum = es, ws
            while len(esum) > 1:
                esum = [a + b for a, b in zip(esum[::2], esum[1::2])]
                wsum = [a + b for a, b in zip(wsum[::2], wsum[1::2])]

            @pl.when(idv[0] == idv[U - 1])
            def _fast():
                off = idv[0] * L
                plsc.addupdate(acc_e.at[pl.ds(off, L)], esum[0])
                plsc.addupdate(acc_w.at[pl.ds(off, L)], wsum[0])
                plsc.addupdate(acc_c.at[pl.ds(off, L)], sixteen)

            @pl.when(idv[0] != idv[U - 1])
            def _slow():
                for u in range(U):
                    off = idv[u] * L
                    plsc.addupdate(acc_e.at[pl.ds(off, L)], es[u])
                    plsc.addupdate(acc_w.at[pl.ds(off, L)], ws[u])
                    plsc.addupdate(acc_c.at[pl.ds(off, L)], ones)

        def row_body(jj, inner):
            base = jj * (4 * U)
            group(base)
            group(base + U)
            group(base + 2 * U)
            group(base + 3 * U)
            return inner

        lax.fori_loop(0, CH // (4 * U), row_body, 0)
        group(CH - U)
        return carry

    lax.fori_loop(0, NCH, chunk_body, 0)

    def fin_body(b, carry):
        off = b * L
        e = acc_e[pl.ds(off, L)]
        wv = acc_w[pl.ds(off, L)]
        n = acc_c[pl.ds(off, L)]
        pooled = wv / e
        out_v = pooled * n / (1.0 + beta_vec * (n - 1.0)) * scale_vec
        obuf[b, :] = out_v
        return carry

    lax.fori_loop(0, B, fin_body, 0)
    pltpu.sync_copy(obuf, out_hbm.at[:, pl.ds(col0, L)])


@jax.jit
def _pool(x, ids, consts):
    mesh = plsc.VectorSubcoreMesh(core_axis_name="c", subcore_axis_name="s")
    f = functools.partial(
        pl.kernel,
        out_type=jax.ShapeDtypeStruct((B, F), jnp.float32),
        mesh=mesh,
        compiler_params=pltpu.CompilerParams(
            use_tc_tiling_on_sc=False, needs_layout_passes=False),
        scratch_types=[
            pltpu.VMEM((2, CH, L), jnp.float32),
            pltpu.VMEM((2 * CH,), jnp.int32),
            pltpu.VMEM((3, L), jnp.float32),
            pltpu.VMEM((B * L,), jnp.float32),
            pltpu.VMEM((B * L,), jnp.float32),
            pltpu.VMEM((B * L,), jnp.float32),
            pltpu.VMEM((B, L), jnp.float32),
            pltpu.SemaphoreType.DMA((2,)),
            pltpu.SemaphoreType.DMA((2,)),
        ],
    )(_pool_body)
    return f(x, ids, consts)


def kernel(x, batch, p, beta, bsize):
    scale = jnp.asarray(bsize, jnp.float32) / jnp.float32(B)
    consts = jnp.stack([
        jnp.broadcast_to(p.astype(jnp.float32), (L,)),
        jnp.broadcast_to(beta.astype(jnp.float32), (L,)),
        jnp.broadcast_to(scale, (L,)),
    ])
    return _pool(x, batch.astype(jnp.int32), consts)


# speedup vs baseline: 1.0086x; 1.0035x over previous
import functools

import jax
import jax.numpy as jnp
from jax import lax
from jax.experimental import pallas as pl
from jax.experimental.pallas import tpu as pltpu
from jax.experimental.pallas import tpu_sc as plsc

N = 50000
F = 512
B = 256
L = 16
NC = 2
NS = 16
CH = 2000
NCH = N // CH
U = 16


def _pool_body(x_hbm, ids_hbm, c_hbm, out_hbm,
               xbuf, idbuf, cbuf, acc_e, acc_w, acc_c, obuf, semx, semi):
    cid = lax.axis_index("c")
    sid = lax.axis_index("s")
    w = sid * NC + cid
    col0 = w * L

    pltpu.sync_copy(c_hbm, cbuf)
    p_vec = cbuf[0, :]
    beta_vec = cbuf[1, :]
    scale_vec = cbuf[2, :]
    zeros = jnp.zeros((L,), jnp.float32)
    ones = jnp.ones((L,), jnp.float32)
    sixteen = jnp.full((L,), float(U), jnp.float32)

    def zero_body(b, carry):
        off = b * L
        acc_e[pl.ds(off, L)] = zeros
        acc_w[pl.ds(off, L)] = zeros
        acc_c[pl.ds(off, L)] = zeros
        return carry

    lax.fori_loop(0, B, zero_body, 0)

    def copies(ci):
        s = lax.rem(ci, 2)
        r0 = ci * CH
        cx = pltpu.make_async_copy(
            x_hbm.at[pl.ds(r0, CH), pl.ds(col0, L)], xbuf.at[s], semx.at[s])
        cb = pltpu.make_async_copy(
            ids_hbm.at[pl.ds(r0, CH)], idbuf.at[pl.ds(s * CH, CH)], semi.at[s])
        return cx, cb

    cx0, cb0 = copies(0)
    cx0.start()
    cb0.start()

    def chunk_body(ci, carry):
        s = lax.rem(ci, 2)
        cx, cb = copies(ci)
        cx.wait()
        cb.wait()

        @pl.when(ci + 1 < NCH)
        def _prefetch():
            nx, nb = copies(ci + 1)
            nx.start()
            nb.start()

        def group(base):
            idv = idbuf[pl.ds(s * CH + base, U)]
            xs = [xbuf[s, base + u, :] for u in range(U)]
            es = [jnp.exp(xv * p_vec) for xv in xs]
            ws = [xs[u] * es[u] for u in range(U)]
            esum, wsum = es, ws
            while len(esum) > 1:
                esum = [a + b for a, b in zip(esum[::2], esum[1::2])]
                wsum = [a + b for a, b in zip(wsum[::2], wsum[1::2])]

            @pl.when(idv[0] == idv[U - 1])
            def _fast():
                off = idv[0] * L
                plsc.addupdate(acc_e.at[pl.ds(off, L)], esum[0])
                plsc.addupdate(acc_w.at[pl.ds(off, L)], wsum[0])
                plsc.addupdate(acc_c.at[pl.ds(off, L)], sixteen)

            @pl.when(idv[0] != idv[U - 1])
            def _slow():
                for u in range(U):
                    off = idv[u] * L
                    plsc.addupdate(acc_e.at[pl.ds(off, L)], es[u])
                    plsc.addupdate(acc_w.at[pl.ds(off, L)], ws[u])
                    plsc.addupdate(acc_c.at[pl.ds(off, L)], ones)

        def row_body(jj, inner):
            base = jj * (5 * U)
            for g in range(5):
                group(base + g * U)
            return inner

        lax.fori_loop(0, CH // (5 * U), row_body, 0)
        return carry

    lax.fori_loop(0, NCH, chunk_body, 0)

    def fin_body(b, carry):
        off = b * L
        e = acc_e[pl.ds(off, L)]
        wv = acc_w[pl.ds(off, L)]
        n = acc_c[pl.ds(off, L)]
        pooled = wv / e
        out_v = pooled * n / (1.0 + beta_vec * (n - 1.0)) * scale_vec
        obuf[b, :] = out_v
        return carry

    lax.fori_loop(0, B, fin_body, 0)
    pltpu.sync_copy(obuf, out_hbm.at[:, pl.ds(col0, L)])


@jax.jit
def _pool(x, ids, consts):
    mesh = plsc.VectorSubcoreMesh(core_axis_name="c", subcore_axis_name="s")
    f = functools.partial(
        pl.kernel,
        out_type=jax.ShapeDtypeStruct((B, F), jnp.float32),
        mesh=mesh,
        compiler_params=pltpu.CompilerParams(
            use_tc_tiling_on_sc=False, needs_layout_passes=False),
        scratch_types=[
            pltpu.VMEM((2, CH, L), jnp.float32),
            pltpu.VMEM((2 * CH,), jnp.int32),
            pltpu.VMEM((3, L), jnp.float32),
            pltpu.VMEM((B * L,), jnp.float32),
            pltpu.VMEM((B * L,), jnp.float32),
            pltpu.VMEM((B * L,), jnp.float32),
            pltpu.VMEM((B, L), jnp.float32),
            pltpu.SemaphoreType.DMA((2,)),
            pltpu.SemaphoreType.DMA((2,)),
        ],
    )(_pool_body)
    return f(x, ids, consts)


def kernel(x, batch, p, beta, bsize):
    scale = jnp.asarray(bsize, jnp.float32) / jnp.float32(B)
    consts = jnp.stack([
        jnp.broadcast_to(p.astype(jnp.float32), (L,)),
        jnp.broadcast_to(beta.astype(jnp.float32), (L,)),
        jnp.broadcast_to(scale, (L,)),
    ])
    return _pool(x, batch.astype(jnp.int32), consts)
